# baseline (speedup 1.0000x reference)
import jax
import jax.numpy as jnp
from jax.experimental import pallas as pl
from jax.experimental.pallas import tpu as pltpu

B, D, STRIPE, NS, K = 4096, 2048, 128, 32, 8
H = NS * STRIPE
BT = 512


def _body(xb_ref, we_hbm, be_ref, wd_hbm, bd_ref, wr_ref, br_ref,
          out_ref, we_v, wd_v, sem_e, sem_d):
    i = pl.program_id(0)

    @pl.when(i == 0)
    def _():
        pltpu.make_async_copy(we_hbm, we_v, sem_e).start()
        pltpu.make_async_copy(wd_hbm, wd_v, sem_d).start()

    scores = jnp.dot(xb_ref[...], wr_ref[...],
                     preferred_element_type=jnp.float32)
    scores = scores + br_ref[...]
    cnt = jnp.zeros_like(scores)
    for j2 in range(NS):
        cnt += (scores[:, j2:j2 + 1] > scores).astype(jnp.float32)
    maskb = (cnt < float(K)).astype(jnp.bfloat16)
    rows = jax.lax.broadcasted_iota(jnp.int32, (NS, H), 0)
    cols = jax.lax.broadcasted_iota(jnp.int32, (NS, H), 1)
    r = (rows == cols // STRIPE).astype(jnp.bfloat16)
    mexp = jnp.dot(maskb, r, preferred_element_type=jnp.float32)

    @pl.when(i == 0)
    def _():
        pltpu.make_async_copy(we_hbm, we_v, sem_e).wait()

    e = jnp.dot(xb_ref[...], we_v[...], preferred_element_type=jnp.float32)
    e = jnp.maximum(e + be_ref[...], 0.0) * mexp
    code = e.astype(jnp.bfloat16)

    @pl.when(i == 0)
    def _():
        pltpu.make_async_copy(wd_hbm, wd_v, sem_d).wait()

    part = jnp.dot(code, wd_v[...], preferred_element_type=jnp.float32)
    out_ref[...] = jnp.maximum(part + bd_ref[...], 0.0)


def _run(xb, we, be2, wd, bd2, wr, br2, interpret=False):
    grid = (B // BT,)
    return pl.pallas_call(
        _body,
        grid=grid,
        in_specs=[
            pl.BlockSpec((BT, D), lambda i: (i, 0)),
            pl.BlockSpec(memory_space=pl.ANY),
            pl.BlockSpec((1, H), lambda i: (0, 0)),
            pl.BlockSpec(memory_space=pl.ANY),
            pl.BlockSpec((1, D), lambda i: (0, 0)),
            pl.BlockSpec((D, NS), lambda i: (0, 0)),
            pl.BlockSpec((1, NS), lambda i: (0, 0)),
        ],
        out_specs=pl.BlockSpec((BT, D), lambda i: (i, 0)),
        out_shape=jax.ShapeDtypeStruct((B, D), jnp.float32),
        scratch_shapes=[
            pltpu.VMEM((D, H), jnp.bfloat16),
            pltpu.VMEM((H, D), jnp.bfloat16),
            pltpu.SemaphoreType.DMA,
            pltpu.SemaphoreType.DMA,
        ],
        compiler_params=pltpu.CompilerParams(
            dimension_semantics=("arbitrary",),
        ),
        interpret=interpret,
    )(xb, we, be2, wd, bd2, wr, br2)


def kernel(x, W_enc, b_enc, W_dec, b_dec, W_rout, b_rout):
    xb = x.astype(jnp.bfloat16)
    we = W_enc.astype(jnp.bfloat16)
    wd = W_dec.astype(jnp.bfloat16)
    wr = W_rout.astype(jnp.bfloat16)
    be2 = b_enc.reshape(1, H)
    bd2 = b_dec.reshape(1, D)
    br2 = b_rout.reshape(1, NS)
    return _run(xb, we, be2, wd, bd2, wr, br2)


# speedup vs baseline: 1.0680x; 1.0680x over previous
import jax
import jax.numpy as jnp
from jax.experimental import pallas as pl
from jax.experimental.pallas import tpu as pltpu

B, D, STRIPE, NS, K = 4096, 2048, 128, 32, 8
H = NS * STRIPE
BT = 512


def _body(xb_ref, we_hbm, be_ref, wd_hbm, bd_ref, wr_ref, br_ref,
          out_ref, we_v, wd_v, sem_e, sem_d):
    i = pl.program_id(0)

    @pl.when(i == 0)
    def _():
        pltpu.make_async_copy(we_hbm, we_v, sem_e).start()
        pltpu.make_async_copy(wd_hbm, wd_v, sem_d).start()

    scores = jnp.dot(xb_ref[...], wr_ref[...],
                     preferred_element_type=jnp.float32)
    scores = scores + br_ref[...]
    cur = scores
    for _ in range(K - 1):
        m = jnp.max(cur, axis=1, keepdims=True)
        cur = jnp.where(cur == m, -jnp.inf, cur)
    thr = jnp.max(cur, axis=1, keepdims=True)
    maskb = (scores >= thr).astype(jnp.bfloat16)
    rows = jax.lax.broadcasted_iota(jnp.int32, (NS, H), 0)
    cols = jax.lax.broadcasted_iota(jnp.int32, (NS, H), 1)
    r = (rows == cols // STRIPE).astype(jnp.bfloat16)
    mexp = jnp.dot(maskb, r, preferred_element_type=jnp.float32)

    @pl.when(i == 0)
    def _():
        pltpu.make_async_copy(we_hbm, we_v, sem_e).wait()

    e = jnp.dot(xb_ref[...], we_v[...], preferred_element_type=jnp.float32)
    e = jnp.maximum(e + be_ref[...], 0.0) * mexp
    code = e.astype(jnp.bfloat16)

    @pl.when(i == 0)
    def _():
        pltpu.make_async_copy(wd_hbm, wd_v, sem_d).wait()

    part = jnp.dot(code, wd_v[...], preferred_element_type=jnp.float32)
    out_ref[...] = jnp.maximum(part + bd_ref[...], 0.0)


def _run(xb, we, be2, wd, bd2, wr, br2, interpret=False):
    grid = (B // BT,)
    return pl.pallas_call(
        _body,
        grid=grid,
        in_specs=[
            pl.BlockSpec((BT, D), lambda i: (i, 0)),
            pl.BlockSpec(memory_space=pl.ANY),
            pl.BlockSpec((1, H), lambda i: (0, 0)),
            pl.BlockSpec(memory_space=pl.ANY),
            pl.BlockSpec((1, D), lambda i: (0, 0)),
            pl.BlockSpec((D, NS), lambda i: (0, 0)),
            pl.BlockSpec((1, NS), lambda i: (0, 0)),
        ],
        out_specs=pl.BlockSpec((BT, D), lambda i: (i, 0)),
        out_shape=jax.ShapeDtypeStruct((B, D), jnp.float32),
        scratch_shapes=[
            pltpu.VMEM((D, H), jnp.bfloat16),
            pltpu.VMEM((H, D), jnp.bfloat16),
            pltpu.SemaphoreType.DMA,
            pltpu.SemaphoreType.DMA,
        ],
        compiler_params=pltpu.CompilerParams(
            dimension_semantics=("arbitrary",),
        ),
        interpret=interpret,
    )(xb, we, be2, wd, bd2, wr, br2)


def kernel(x, W_enc, b_enc, W_dec, b_dec, W_rout, b_rout):
    xb = x.astype(jnp.bfloat16)
    we = W_enc.astype(jnp.bfloat16)
    wd = W_dec.astype(jnp.bfloat16)
    wr = W_rout.astype(jnp.bfloat16)
    be2 = b_enc.reshape(1, H)
    bd2 = b_dec.reshape(1, D)
    br2 = b_rout.reshape(1, NS)
    return _run(xb, we, be2, wd, bd2, wr, br2)
